# baseline (speedup 1.0000x reference)
import functools

import jax
import jax.numpy as jnp
from jax import lax
from jax.experimental import pallas as pl
from jax.experimental.pallas import tpu as pltpu
from jax.experimental.pallas import tpu_sc as plsc

N = 10000
NP = 10240
E = 320000
H = 128
K = 1544
TRASH = 10000
NTR = 128
BM = 128
GM = 79
NT = 16
CPB = 16
NB = 10
NCH = CPB * NB
EP = NT * NCH * 128
EROW = EP // 128
ROWS_PER_TILE = NP // NT


def _leaky(v):
    return jnp.where(v > 0, v, 0.01 * v)


def _h0_body(x_ref, wbig_ref, bbig_ref, win_ref, bin_ref, pa_ref, h_ref):
    x = x_ref[...]
    h1 = jnp.dot(x, wbig_ref[...], preferred_element_type=jnp.float32)
    h1 = _leaky(h1 + bbig_ref[...])
    h = jnp.dot(h1, win_ref[...], preferred_element_type=jnp.float32)
    h = h + bin_ref[...]
    h_ref[...] = jnp.where(h > 0, h, pa_ref[...] * h)


def _expand_body(h_ref, w_ref, b_ref, yy_ref):
    yy_ref[...] = (
        jnp.dot(h_ref[...], w_ref[0], preferred_element_type=jnp.float32)
        + b_ref[0])


def _combine(root, s, cnt):
    c0 = jnp.maximum(cnt[0, :, 0:1], 1.0)
    c1 = jnp.maximum(cnt[1, :, 0:1], 1.0)
    return root + s[0] / c0 + s[1] / c1


def _mid_body(yy_ref, s_ref, cnt_ref, h_ref):
    h_ref[...] = _combine(yy_ref[...], s_ref[...], cnt_ref[...])


def _final_body(yy_ref, s_ref, cnt_ref, wcls_ref, bcls_ref, out_ref):
    h = _combine(yy_ref[...], s_ref[...], cnt_ref[...])
    out_ref[...] = (
        jnp.dot(h, wcls_ref[...], preferred_element_type=jnp.float32)
        + bcls_ref[...])


def _prep_body(src_ref, dst_ref, et_ref, g_ref, s_ref):
    s = src_ref[...]
    d = dst_ref[...]
    t = et_ref[...]
    trash = TRASH + lax.broadcasted_iota(jnp.int32, s.shape, 1)
    g_ref[0] = s + NP
    g_ref[1] = s + 2 * NP
    s_ref[0] = jnp.where(t == 0, d, trash)
    s_ref[1] = jnp.where(t == 1, d, trash)


def _FULL(shape):
    return pl.BlockSpec(shape, lambda *_: tuple(0 for _ in shape))


def _ROWB(nd=1):
    if nd == 1:
        return pl.BlockSpec((BM, H), lambda i: (i, 0))
    return pl.BlockSpec((2, BM, H), lambda i: (0, i, 0))


def _h0(x, wbig, bbig, win, b_in, pa):
    return pl.pallas_call(
        _h0_body,
        grid=(GM,),
        in_specs=[
            pl.BlockSpec((BM, K), lambda i: (i, 0)),
            _FULL((K, H)), _FULL((1, H)), _FULL((H, H)), _FULL((1, H)),
            _FULL((1, H)),
        ],
        out_specs=_ROWB(),
        out_shape=jax.ShapeDtypeStruct((N, H), jnp.float32),
    )(x, wbig, bbig, win, b_in, pa)


def _expand(h, wstack, bstack):
    return pl.pallas_call(
        _expand_body,
        grid=(3, GM),
        in_specs=[
            pl.BlockSpec((BM, H), lambda k, i: (i, 0)),
            pl.BlockSpec((1, H, H), lambda k, i: (k, 0, 0)),
            pl.BlockSpec((1, 1, H), lambda k, i: (k, 0, 0)),
        ],
        out_specs=pl.BlockSpec((BM, H), lambda k, i: (k * (NP // BM) + i, 0)),
        out_shape=jax.ShapeDtypeStruct((3 * NP, H), jnp.float32),
    )(h, wstack, bstack)


def _mid(yy, s, cnt):
    return pl.pallas_call(
        _mid_body,
        grid=(GM,),
        in_specs=[_ROWB(), _ROWB(2), _ROWB(2)],
        out_specs=_ROWB(),
        out_shape=jax.ShapeDtypeStruct((N, H), jnp.float32),
    )(yy, s, cnt)


def _final(yy, s, cnt, wcls, bcls):
    return pl.pallas_call(
        _final_body,
        grid=(GM,),
        in_specs=[_ROWB(), _ROWB(2), _ROWB(2), _FULL((H, H)), _FULL((1, H))],
        out_specs=_ROWB(),
        out_shape=jax.ShapeDtypeStruct((N, H), jnp.float32),
    )(yy, s, cnt, wcls, bcls)


def _prep(src2d, dst2d, et2d):
    return pl.pallas_call(
        _prep_body,
        grid=(NCH,),
        in_specs=[pl.BlockSpec((EROW // NCH, 128), lambda i: (i, 0))] * 3,
        out_specs=[pl.BlockSpec((2, EROW // NCH, 128),
                                lambda i: (0, i, 0))] * 2,
        out_shape=[jax.ShapeDtypeStruct((2, EROW, 128), jnp.int32)] * 2,
    )(src2d, dst2d, et2d)


def _zero_buf(buf):
    def _zrow(i, carry):
        for j in range(H // 16):
            buf[i, pl.ds(j * 16, 16)] = jnp.zeros((16,), jnp.float32)
        return carry
    lax.fori_loop(0, 128, _zrow, 0)


def _zero_acc(buf, acc, row0):
    def _zcopy(k, carry):
        pltpu.sync_copy(buf, acc.at[pl.ds(row0 + k * 128, 128)])
        return carry
    lax.fori_loop(0, ROWS_PER_TILE // 128, _zcopy, 0)


def _write_out(buf, acc, out_hbm, c, row0):
    def _obody(k, carry):
        pltpu.sync_copy(acc.at[pl.ds(row0 + k * 128, 128)], buf)
        pltpu.sync_copy(buf, out_hbm.at[c, pl.ds(row0 + k * 128, 128)])
        return carry
    lax.fori_loop(0, ROWS_PER_TILE // 128, _obody, 0)


def _sc_agg_body(yy_hbm, gidx_hbm, sidx_hbm, out_hbm,
                 gidx_v, sidx_v, buf0, buf1, acc, sem):
    c = lax.axis_index("c")
    s = lax.axis_index("s")
    row0 = s * ROWS_PER_TILE
    erow0 = s * NB * CPB

    _zero_buf(buf0)
    _zero_acc(buf0, acc, row0)
    plsc.subcore_barrier()

    def _block(nb, carry):
        pltpu.sync_copy(gidx_hbm.at[c, pl.ds(erow0 + nb * CPB, CPB)], gidx_v)
        pltpu.sync_copy(sidx_hbm.at[c, pl.ds(erow0 + nb * CPB, CPB)], sidx_v)
        pltpu.async_copy(yy_hbm.at[gidx_v.at[0]], buf0, sem)

        def _mbody(g, carry2):
            for b in range(2):
                bufa = buf0 if b == 0 else buf1
                bufb = buf1 if b == 0 else buf0
                j = g * 2 + b
                pltpu.make_async_copy(yy_hbm.at[gidx_v.at[j]], bufa,
                                      sem).wait()
                jn = jnp.minimum(j + 1, CPB - 1)
                pltpu.async_copy(yy_hbm.at[gidx_v.at[jn]], bufb, sem)
                pltpu.sync_copy(bufa, acc.at[sidx_v.at[j]], add=True)
            return carry2
        lax.fori_loop(0, CPB // 2, _mbody, 0)
        pltpu.make_async_copy(yy_hbm.at[gidx_v.at[0]], buf0, sem).wait()
        return carry
    lax.fori_loop(0, NB, _block, 0)
    plsc.subcore_barrier()

    _write_out(buf1, acc, out_hbm, c, row0)


def _sc_cnt_body(sidx_hbm, out_hbm, sidx_v, buf0, ones_v, acc):
    c = lax.axis_index("c")
    s = lax.axis_index("s")
    row0 = s * ROWS_PER_TILE
    erow0 = s * NB * CPB

    _zero_buf(buf0)
    e0 = jnp.where(lax.broadcasted_iota(jnp.int32, (16,), 0) == 0, 1.0, 0.0)

    def _orow(i, carry):
        ones_v[i, pl.ds(0, 16)] = e0
        for j in range(1, H // 16):
            ones_v[i, pl.ds(j * 16, 16)] = jnp.zeros((16,), jnp.float32)
        return carry
    lax.fori_loop(0, 128, _orow, 0)

    _zero_acc(buf0, acc, row0)
    plsc.subcore_barrier()

    def _block(nb, carry):
        pltpu.sync_copy(sidx_hbm.at[c, pl.ds(erow0 + nb * CPB, CPB)], sidx_v)

        def _mbody(j, carry2):
            pltpu.sync_copy(ones_v, acc.at[sidx_v.at[j]], add=True)
            return carry2
        lax.fori_loop(0, CPB, _mbody, 0)
        return carry
    lax.fori_loop(0, NB, _block, 0)
    plsc.subcore_barrier()

    _write_out(buf0, acc, out_hbm, c, row0)


def _sc_mesh():
    return plsc.VectorSubcoreMesh(
        core_axis_name="c", subcore_axis_name="s",
        num_cores=2, num_subcores=NT)


@functools.cache
def _make_sc_agg():
    return pl.kernel(
        _sc_agg_body,
        out_type=jax.ShapeDtypeStruct((2, NP, H), jnp.float32),
        mesh=_sc_mesh(),
        scratch_types=[
            pltpu.VMEM((CPB, 128), jnp.int32),
            pltpu.VMEM((CPB, 128), jnp.int32),
            pltpu.VMEM((128, H), jnp.float32),
            pltpu.VMEM((128, H), jnp.float32),
            pltpu.VMEM_SHARED((NP, H), jnp.float32),
            pltpu.SemaphoreType.DMA,
        ],
    )


@functools.cache
def _make_sc_cnt():
    return pl.kernel(
        _sc_cnt_body,
        out_type=jax.ShapeDtypeStruct((2, NP, H), jnp.float32),
        mesh=_sc_mesh(),
        scratch_types=[
            pltpu.VMEM((CPB, 128), jnp.int32),
            pltpu.VMEM((128, H), jnp.float32),
            pltpu.VMEM((128, H), jnp.float32),
            pltpu.VMEM_SHARED((NP, H), jnp.float32),
        ],
    )


def kernel(x, edge_index, edge_type, W_des, b_des, W_tweet, b_tweet,
           W_num, b_num, W_cat, b_cat, W_in, b_in, prelu_a,
           W_rel1, W_root1, b1, W_rel2, W_root2, b2, W_cls, b_cls):
    f32 = jnp.float32
    D_NUM, D_TWEET, D_CAT = 5, 768, 3

    wbig = jnp.zeros((K, H), f32)
    wbig = wbig.at[0:D_NUM, 64:96].set(W_num)
    wbig = wbig.at[D_NUM:D_NUM + D_TWEET, 32:64].set(W_tweet)
    wbig = wbig.at[D_NUM + D_TWEET:D_NUM + D_TWEET + D_CAT, 96:128].set(W_cat)
    wbig = wbig.at[D_NUM + D_TWEET + D_CAT:K, 0:32].set(W_des)
    bbig = jnp.concatenate([b_des, b_tweet, b_num, b_cat])[None, :]
    wstack1 = jnp.stack([W_root1, W_rel1[0], W_rel1[1]])
    bstack1 = jnp.stack([b1, jnp.zeros_like(b1), jnp.zeros_like(b1)])[:, None]
    wstack2 = jnp.stack([W_root2, W_rel2[0], W_rel2[1]])
    bstack2 = jnp.stack([b2, jnp.zeros_like(b2), jnp.zeros_like(b2)])[:, None]

    src = jnp.pad(edge_index[0].astype(jnp.int32), (0, EP - E))
    dst = jnp.pad(edge_index[1].astype(jnp.int32), (0, EP - E))
    et = jnp.pad(edge_type.astype(jnp.int32), (0, EP - E),
                 constant_values=2)
    gidx, sidx = _prep(src.reshape(EROW, 128), dst.reshape(EROW, 128),
                       et.reshape(EROW, 128))

    cnts = _make_sc_cnt()(sidx)
    h0 = _h0(x, wbig, bbig, W_in, b_in[None, :], prelu_a[None, :])
    yy1 = _expand(h0, wstack1, bstack1)
    s1 = _make_sc_agg()(yy1, gidx, sidx)
    h1 = _mid(yy1, s1, cnts)
    yy2 = _expand(h1, wstack2, bstack2)
    s2 = _make_sc_agg()(yy2, gidx, sidx)
    return _final(yy2, s2, cnts, W_cls, b_cls[None, :])
